# baseline (speedup 1.0000x reference)
import functools

import jax
import jax.numpy as jnp
from jax import lax
from jax.experimental import pallas as pl
from jax.experimental.pallas import tpu as pltpu
from jax.experimental.pallas import tpu_sc as plsc

NC = 2
NS = 16
NW = NC * NS

NI = 4096
NJ = 200
V = 1_000_000
C = 32
IB = NI // NW
L = 16

_MESH = plsc.VectorSubcoreMesh(
    core_axis_name="c", subcore_axis_name="s", num_cores=NC, num_subcores=NS
)


GJ = 4
NG = NJ // GJ


@functools.partial(
    pl.kernel,
    out_type=jax.ShapeDtypeStruct((NJ * C, NI), jnp.float32),
    mesh=_MESH,
    scratch_types=[
        pltpu.VMEM((NJ, IB), jnp.int32),
        pltpu.VMEM((NG, GJ * IB), jnp.int32),
        [pltpu.VMEM((GJ * IB, C), jnp.float32) for _ in range(2)],
        [[pltpu.VMEM((C, IB), jnp.float32) for _ in range(GJ)]
         for _ in range(2)],
        pltpu.SemaphoreType.DMA,
        [pltpu.SemaphoreType.DMA for _ in range(2)],
        [pltpu.SemaphoreType.DMA for _ in range(2)],
    ],
    compiler_params=pltpu.CompilerParams(use_tc_tiling_on_sc=False, needs_layout_passes=False),
)
def _grid_gather(
    tt_hbm, table_hbm, out_hbm,
    idx_v, idx2_v, rows, slab,
    sem_idx, sem_g, sem_s,
):
    wid = lax.axis_index("s") * NC + lax.axis_index("c")
    i0 = wid * IB
    iota = lax.iota(jnp.int32, L)

    pltpu.async_copy(tt_hbm.at[:, pl.ds(i0, IB)], idx_v, sem_idx).wait()

    def repack(q, carry):
        for jj in range(GJ):
            for k in range(IB // L):
                idx2_v[q, pl.ds(jj * IB + k * L, L)] = idx_v[
                    q * GJ + jj, pl.ds(k * L, L)
                ]
        return carry

    lax.fori_loop(0, NG, repack, 0)

    def fire_group(q, b):
        pltpu.async_copy(table_hbm.at[idx2_v.at[q]], rows[b], sem_g[b])

    fire_group(0, 0)
    fire_group(1, 1)

    def outer(g, carry):
        for b in range(2):
            q = 2 * g + b
            pltpu.make_async_copy(
                table_hbm.at[idx2_v.at[q]], rows[b], sem_g[b]
            ).wait()

            @pl.when(q >= 2)
            def _():
                for jj in range(GJ):
                    pltpu.make_async_copy(
                        slab[b][jj],
                        out_hbm.at[pl.ds((q * GJ + jj) * C, C), pl.ds(i0, IB)],
                        sem_s[b],
                    ).wait()

            for jj in range(GJ):

                def tbody(k, carry2, jj=jj):
                    rowvec = jj * IB + k * L + iota
                    for c in range(C):
                        slab[b][jj][c, pl.ds(k * L, L)] = plsc.load_gather(
                            rows[b], [rowvec, jnp.full((L,), c, jnp.int32)]
                        )
                    return carry2

                lax.fori_loop(0, IB // L, tbody, 0)

            @pl.when(q + 2 < NG)
            def _():
                fire_group(q + 2, b)

            for jj in range(GJ):
                pltpu.async_copy(
                    slab[b][jj],
                    out_hbm.at[pl.ds((q * GJ + jj) * C, C), pl.ds(i0, IB)],
                    sem_s[b],
                )
        return carry

    lax.fori_loop(0, NG // 2, outer, 0)

    for b in range(2):
        for jj in range(GJ):
            pltpu.make_async_copy(
                slab[b][jj],
                out_hbm.at[pl.ds(((NG - 2 + b) * GJ + jj) * C, C), pl.ds(i0, IB)],
                sem_s[b],
            ).wait()


def kernel(t, grid):
    tt2 = t.T.astype(jnp.int32)
    out2 = _grid_gather(tt2, grid)
    return out2.reshape(NJ, C, NI).transpose(2, 0, 1)


# speedup vs baseline: 1.2491x; 1.2491x over previous
import functools

import jax
import jax.numpy as jnp
from jax import lax
from jax.experimental import pallas as pl
from jax.experimental.pallas import tpu as pltpu
from jax.experimental.pallas import tpu_sc as plsc

NC = 2
NS = 16
NW = NC * NS

B = 4096 * 200
C = 32
N_PER_W = B // NW
CHUNK = 800
NBUF = 4
N_CHUNKS = N_PER_W // CHUNK
N_OUTER = N_CHUNKS // NBUF

_MESH = plsc.VectorSubcoreMesh(
    core_axis_name="c", subcore_axis_name="s", num_cores=NC, num_subcores=NS
)


@functools.partial(
    pl.kernel,
    out_type=jax.ShapeDtypeStruct((B, C), jnp.float32),
    mesh=_MESH,
    scratch_types=[
        pltpu.VMEM((N_PER_W,), jnp.int32),
        [pltpu.VMEM((CHUNK, C), jnp.float32) for _ in range(NBUF)],
        [pltpu.SemaphoreType.DMA for _ in range(NBUF)],
    ],
    compiler_params=pltpu.CompilerParams(use_tc_tiling_on_sc=False),
)
def _grid_gather(idx_hbm, table_hbm, out_hbm, idx_v, rows, sems):
    wid = lax.axis_index("s") * NC + lax.axis_index("c")
    base = wid * N_PER_W

    pltpu.sync_copy(idx_hbm.at[pl.ds(base, N_PER_W)], idx_v)

    def fire(chunk, b):
        pltpu.async_copy(
            table_hbm.at[idx_v.at[pl.ds(chunk * CHUNK, CHUNK)]], rows[b], sems[b]
        )

    for b in range(NBUF):
        fire(b, b)

    def outer(g, carry):
        first = g * NBUF
        for b in range(NBUF):
            pltpu.make_async_copy(
                table_hbm.at[idx_v.at[pl.ds(0, CHUNK)]], rows[b], sems[b]
            ).wait()
            pltpu.sync_copy(rows[b], out_hbm.at[pl.ds(base + (first + b) * CHUNK, CHUNK)])
            nxt = first + b + NBUF

            @pl.when(nxt < N_CHUNKS)
            def _():
                fire(nxt, b)

        return carry

    lax.fori_loop(0, N_OUTER, outer, 0)


def kernel(t, grid):
    flat_idx = t.reshape(-1).astype(jnp.int32)
    out = _grid_gather(flat_idx, grid)
    return out.reshape(t.shape + (grid.shape[1],))
